# baseline (speedup 1.0000x reference)
import functools

import jax
import jax.numpy as jnp
from jax import lax
from jax.experimental import pallas as pl
from jax.experimental.pallas import tpu as pltpu
from jax.experimental.pallas import tpu_sc as plsc

NC = 2
NS = 16
CH = 128
HALF = 128
NBUF = 2


def _make_segsum(n, npad, epad, with_deg):
    e_tile = epad // NS
    nchunks = e_tile // CH
    assert nchunks % NBUF == 0
    NI = 4
    zrows = npad // NS

    mesh = plsc.VectorSubcoreMesh(core_axis_name="c", subcore_axis_name="s")

    out_type = [
        jax.ShapeDtypeStruct((npad, HALF), jnp.float32),
        jax.ShapeDtypeStruct((npad, HALF), jnp.float32),
    ]
    scratch = [
        pltpu.VMEM((16, HALF), jnp.float32),
        pltpu.VMEM_SHARED((npad, HALF), jnp.float32),
    ]
    scratch += [pltpu.VMEM((CH,), jnp.int32) for _ in range(2 * NI)]
    scratch += [pltpu.SemaphoreType.DMA for _ in range(NI)]
    scratch += [pltpu.VMEM((CH, HALF), jnp.float32) for _ in range(NBUF)]
    scratch += [pltpu.SemaphoreType.DMA for _ in range(NBUF)]
    scratch += [pltpu.SemaphoreType.DMA for _ in range(NBUF)]
    scratch += [pltpu.SemaphoreType.DMA for _ in range(NBUF)]
    if with_deg:
        out_type.append(jax.ShapeDtypeStruct((npad,), jnp.float32))
        scratch += [
            pltpu.VMEM((CH,), jnp.float32),
            pltpu.VMEM((zrows,), jnp.float32),
            pltpu.VMEM_SHARED((npad,), jnp.float32),
        ]

    def body(tab, src_h, dst_h, out_lo, out_hi, *rest):
        if with_deg:
            deg_out = rest[0]
            rest = rest[1:]
        zb_v, acc = rest[:2]
        sv = rest[2:2 + NI]
        dv = rest[2 + NI:2 + 2 * NI]
        isems = rest[2 + 2 * NI:2 + 3 * NI]
        rows = rest[2 + 3 * NI:2 + 3 * NI + NBUF]
        gsems = rest[2 + 3 * NI + NBUF:2 + 3 * NI + 2 * NBUF]
        ssems = rest[2 + 3 * NI + 2 * NBUF:2 + 3 * NI + 3 * NBUF]
        dsems = rest[2 + 3 * NI + 3 * NBUF:2 + 3 * NI + 4 * NBUF]
        if with_deg:
            ones_v, zd_v, dacc = rest[2 + 3 * NI + 4 * NBUF:]
        cid = lax.axis_index("c")
        sid = lax.axis_index("s")

        zv = jnp.zeros((16,), jnp.float32)
        for r in range(16):
            for g in range(HALF // 16):
                zb_v[r, pl.ds(g * 16, 16)] = zv
        if with_deg:
            ov = jnp.full((16,), 1.0, jnp.float32)
            for g in range(CH // 16):
                ones_v[pl.ds(g * 16, 16)] = ov
            for g in range(zrows // 16):
                zd_v[pl.ds(g * 16, 16)] = zv

        sbase = sid * e_tile
        dbase = sid * e_tile
        coff = cid * HALF

        def issue_idx(i, k):
            pltpu.async_copy(src_h.at[pl.ds(sbase + i * CH, CH)], sv[k], isems[k])
            pltpu.async_copy(dst_h.at[pl.ds(dbase + i * CH, CH)], dv[k], isems[k])

        def wait_idx(i, k):
            pltpu.make_async_copy(src_h.at[pl.ds(sbase + i * CH, CH)], sv[k],
                                  isems[k]).wait()
            pltpu.make_async_copy(dst_h.at[pl.ds(dbase + i * CH, CH)], dv[k],
                                  isems[k]).wait()

        def issue_g(i, k, kr):
            pltpu.async_copy(tab.at[sv[k], pl.ds(coff, HALF)], rows[kr],
                             gsems[kr])

        def wait_g(i, kr):
            pltpu.make_async_copy(tab.at[sv[0], pl.ds(coff, HALF)], rows[kr],
                                  gsems[kr]).wait()

        for b in range(NI):
            issue_idx(b, b)

        @pl.loop(0, zrows // 16)
        def _(i):
            pltpu.sync_copy(zb_v, acc.at[pl.ds(sid * zrows + i * 16, 16)])

        if with_deg:
            pltpu.sync_copy(zd_v, dacc.at[pl.ds(sid * zrows, zrows)])

        wait_idx(0, 0)
        issue_g(0, 0, 0)
        plsc.subcore_barrier()

        def issue_s(k, kr, kb):
            pltpu.async_copy(rows[kr], acc.at[dv[k]], ssems[kb], add=True)
            if with_deg:
                @pl.when(cid == 0)
                def _():
                    pltpu.async_copy(ones_v, dacc.at[dv[k]], dsems[kb], add=True)

        def wait_s(k, kr, kb):
            pltpu.make_async_copy(rows[kr], acc.at[dv[k]], ssems[kb]).wait()
            if with_deg:
                @pl.when(cid == 0)
                def _():
                    pltpu.make_async_copy(ones_v, dacc.at[dv[k]],
                                          dsems[kb]).wait()

        @pl.loop(0, nchunks // NI)
        def _(t):
            for k in range(NI):
                i = t * NI + k

                @pl.when(i - 1 >= 0)
                def _():
                    wait_s((k - 1) % NI, (k - 1) % NBUF, (k - 1) % NBUF)

                    @pl.when(i - 1 + NI < nchunks)
                    def _():
                        issue_idx(i - 1 + NI, (k - 1) % NI)

                @pl.when(i + 1 < nchunks)
                def _():
                    wait_idx(i + 1, (k + 1) % NI)
                    issue_g(i + 1, (k + 1) % NI, (k + 1) % NBUF)

                wait_g(i, k % NBUF)
                issue_s(k, k % NBUF, k % NBUF)

        wait_s((nchunks - 1) % NI, (nchunks - 1) % NBUF, (nchunks - 1) % NBUF)

        plsc.subcore_barrier()

        ob = sid * zrows

        @pl.when(cid == 0)
        def _():
            pltpu.sync_copy(acc.at[pl.ds(ob, zrows)], out_lo.at[pl.ds(ob, zrows)])

        @pl.when(cid == 1)
        def _():
            pltpu.sync_copy(acc.at[pl.ds(ob, zrows)], out_hi.at[pl.ds(ob, zrows)])

        if with_deg:
            @pl.when(cid == 0)
            def _():
                pltpu.sync_copy(dacc.at[pl.ds(sid * zrows, zrows)],
                                deg_out.at[pl.ds(sid * zrows, zrows)])

    return pl.kernel(body, out_type=out_type, mesh=mesh, scratch_types=scratch)


def _selfproj_body(x_ref, w_ref, b_ref, out_ref):
    out_ref[...] = jnp.dot(x_ref[...], w_ref[...],
                           preferred_element_type=jnp.float32) + b_ref[...]


def _mid_body(a_ref, slo_ref, shi_ref, deg_ref, wn1_ref, wn2_ref,
              h_ref, p_ref):
    invd = (1.0 / jnp.maximum(deg_ref[...], 1.0))[:, None]
    mean = jnp.concatenate([slo_ref[...], shi_ref[...]], axis=1) * invd
    h = jnp.maximum(
        a_ref[...] + jnp.dot(mean.astype(jnp.bfloat16), wn1_ref[...],
                             preferred_element_type=jnp.float32), 0.0)
    h_ref[...] = h
    p_ref[...] = jnp.dot(h.astype(jnp.bfloat16), wn2_ref[...],
                         preferred_element_type=jnp.float32)


def _fin_body(s2_ref, slo_ref, shi_ref, deg_ref, out_ref):
    invd = (1.0 / jnp.maximum(deg_ref[...], 1.0))[:, None]
    out_ref[...] = s2_ref[...] + jnp.concatenate(
        [slo_ref[...], shi_ref[...]], axis=1) * invd


def kernel(x, edge_index, W_self1, W_neigh1, b1, W_self2, W_neigh2, b2):
    n, d_in = x.shape
    d_hid = W_self1.shape[1]
    d_out = W_self2.shape[1]
    e = edge_index.shape[1]
    assert d_in == 2 * HALF and d_out == 2 * HALF

    npad = (n // (NS * 16) + 1) * (NS * 16)
    epad = -(-e // (NS * CH * NBUF)) * (NS * CH * NBUF)
    pad = epad - e
    nchunks = epad // NS // CH
    src = edge_index[0]
    dst = edge_index[1]
    if pad:
        fill = jnp.arange(pad, dtype=jnp.int32)
        src = jnp.concatenate([src, fill % n])
        dst = jnp.concatenate([dst, n + fill % (npad - n)])
    segsum_deg = _make_segsum(n, npad, epad, True)
    segsum = _make_segsum(n, npad, epad, False)

    sum1lo, sum1hi, degp = segsum_deg(x, src, dst)

    bm = 1024
    grid = (-(-n // bm),)

    a = pl.pallas_call(
        _selfproj_body,
        grid=grid,
        in_specs=[
            pl.BlockSpec((bm, d_in), lambda i: (i, 0)),
            pl.BlockSpec((d_in, d_hid), lambda i: (0, 0)),
            pl.BlockSpec((1, d_hid), lambda i: (0, 0)),
        ],
        out_specs=pl.BlockSpec((bm, d_hid), lambda i: (i, 0)),
        out_shape=jax.ShapeDtypeStruct((n, d_hid), jnp.float32),
    )(x, W_self1, b1.reshape(1, -1))

    h, p = pl.pallas_call(
        _mid_body,
        grid=grid,
        in_specs=[
            pl.BlockSpec((bm, d_hid), lambda i: (i, 0)),
            pl.BlockSpec((bm, HALF), lambda i: (i, 0)),
            pl.BlockSpec((bm, HALF), lambda i: (i, 0)),
            pl.BlockSpec((bm,), lambda i: (i,)),
            pl.BlockSpec((d_in, d_hid), lambda i: (0, 0)),
            pl.BlockSpec((d_hid, d_out), lambda i: (0, 0)),
        ],
        out_specs=[
            pl.BlockSpec((bm, d_hid), lambda i: (i, 0)),
            pl.BlockSpec((bm, d_out), lambda i: (i, 0)),
        ],
        out_shape=[
            jax.ShapeDtypeStruct((n, d_hid), jnp.float32),
            jax.ShapeDtypeStruct((n, d_out), jnp.float32),
        ],
    )(a, sum1lo, sum1hi, degp, W_neigh1.astype(jnp.bfloat16),
      W_neigh2.astype(jnp.bfloat16))

    sum2lo, sum2hi = segsum(p, src, dst)

    s2 = pl.pallas_call(
        _selfproj_body,
        grid=grid,
        in_specs=[
            pl.BlockSpec((bm, d_hid), lambda i: (i, 0)),
            pl.BlockSpec((d_hid, d_out), lambda i: (0, 0)),
            pl.BlockSpec((1, d_out), lambda i: (0, 0)),
        ],
        out_specs=pl.BlockSpec((bm, d_out), lambda i: (i, 0)),
        out_shape=jax.ShapeDtypeStruct((n, d_out), jnp.float32),
    )(h, W_self2, b2.reshape(1, -1))

    out = pl.pallas_call(
        _fin_body,
        grid=grid,
        in_specs=[
            pl.BlockSpec((bm, d_out), lambda i: (i, 0)),
            pl.BlockSpec((bm, HALF), lambda i: (i, 0)),
            pl.BlockSpec((bm, HALF), lambda i: (i, 0)),
            pl.BlockSpec((bm,), lambda i: (i,)),
        ],
        out_specs=pl.BlockSpec((bm, d_out), lambda i: (i, 0)),
        out_shape=jax.ShapeDtypeStruct((n, d_out), jnp.float32),
    )(s2, sum2lo, sum2hi, degp)

    return out


# speedup vs baseline: 1.0309x; 1.0309x over previous
import functools

import jax
import jax.numpy as jnp
from jax import lax
from jax.experimental import pallas as pl
from jax.experimental.pallas import tpu as pltpu
from jax.experimental.pallas import tpu_sc as plsc

NC = 2
NS = 16
CH = 128
HALF = 128
NBUF = 2


def _make_segsum(n, npad, epad, with_deg):
    e_tile = epad // NS
    nchunks = e_tile // CH
    assert nchunks % NBUF == 0
    NI = 4
    zrows = npad // NS

    mesh = plsc.VectorSubcoreMesh(core_axis_name="c", subcore_axis_name="s")

    out_type = [
        jax.ShapeDtypeStruct((npad, HALF), jnp.float32),
        jax.ShapeDtypeStruct((npad, HALF), jnp.float32),
    ]
    scratch = [
        pltpu.VMEM((16, HALF), jnp.float32),
        pltpu.VMEM_SHARED((npad, HALF), jnp.float32),
    ]
    scratch += [pltpu.VMEM((CH,), jnp.int32) for _ in range(2 * NI)]
    scratch += [pltpu.SemaphoreType.DMA for _ in range(NI)]
    scratch += [pltpu.VMEM((CH, HALF), jnp.float32) for _ in range(NBUF)]
    scratch += [pltpu.SemaphoreType.DMA for _ in range(NBUF)]
    scratch += [pltpu.SemaphoreType.DMA for _ in range(NBUF)]
    scratch += [pltpu.SemaphoreType.DMA for _ in range(NBUF)]
    if with_deg:
        out_type.append(jax.ShapeDtypeStruct((npad,), jnp.float32))
        scratch += [
            pltpu.VMEM((CH,), jnp.float32),
            pltpu.VMEM((zrows,), jnp.float32),
            pltpu.VMEM_SHARED((npad,), jnp.float32),
        ]

    def body(tab, src_h, dst_h, out_lo, out_hi, *rest):
        if with_deg:
            deg_out = rest[0]
            rest = rest[1:]
        zb_v, acc = rest[:2]
        sv = rest[2:2 + NI]
        dv = rest[2 + NI:2 + 2 * NI]
        isems = rest[2 + 2 * NI:2 + 3 * NI]
        rows = rest[2 + 3 * NI:2 + 3 * NI + NBUF]
        gsems = rest[2 + 3 * NI + NBUF:2 + 3 * NI + 2 * NBUF]
        ssems = rest[2 + 3 * NI + 2 * NBUF:2 + 3 * NI + 3 * NBUF]
        dsems = rest[2 + 3 * NI + 3 * NBUF:2 + 3 * NI + 4 * NBUF]
        if with_deg:
            ones_v, zd_v, dacc = rest[2 + 3 * NI + 4 * NBUF:]
        cid = lax.axis_index("c")
        sid = lax.axis_index("s")

        zv = jnp.zeros((16,), jnp.float32)
        for r in range(16):
            for g in range(HALF // 16):
                zb_v[r, pl.ds(g * 16, 16)] = zv
        if with_deg:
            ov = jnp.full((16,), 1.0, jnp.float32)
            for g in range(CH // 16):
                ones_v[pl.ds(g * 16, 16)] = ov
            for g in range(zrows // 16):
                zd_v[pl.ds(g * 16, 16)] = zv

        sbase = sid * e_tile
        dbase = sid * e_tile
        coff = cid * HALF

        def issue_idx(i, k):
            pltpu.async_copy(src_h.at[pl.ds(sbase + i * CH, CH)], sv[k], isems[k])
            pltpu.async_copy(dst_h.at[pl.ds(dbase + i * CH, CH)], dv[k], isems[k])

        def wait_idx(i, k):
            pltpu.make_async_copy(src_h.at[pl.ds(sbase + i * CH, CH)], sv[k],
                                  isems[k]).wait()
            pltpu.make_async_copy(dst_h.at[pl.ds(dbase + i * CH, CH)], dv[k],
                                  isems[k]).wait()

        def issue_g(i, k, kr):
            pltpu.async_copy(tab.at[sv[k], pl.ds(coff, HALF)], rows[kr],
                             gsems[kr])

        def wait_g(i, kr):
            pltpu.make_async_copy(tab.at[sv[0], pl.ds(coff, HALF)], rows[kr],
                                  gsems[kr]).wait()

        for b in range(NI):
            issue_idx(b, b)

        @pl.loop(0, zrows // 16)
        def _(i):
            pltpu.sync_copy(zb_v, acc.at[pl.ds(sid * zrows + i * 16, 16)])

        if with_deg:
            pltpu.sync_copy(zd_v, dacc.at[pl.ds(sid * zrows, zrows)])

        wait_idx(0, 0)
        issue_g(0, 0, 0)
        plsc.subcore_barrier()

        def issue_s(k, kr, kb):
            pltpu.async_copy(rows[kr], acc.at[dv[k]], ssems[kb], add=True)
            if with_deg:
                @pl.when(cid == 0)
                def _():
                    pltpu.async_copy(ones_v, dacc.at[dv[k]], dsems[kb], add=True)

        def wait_s(k, kr, kb):
            pltpu.make_async_copy(rows[kr], acc.at[dv[k]], ssems[kb]).wait()
            if with_deg:
                @pl.when(cid == 0)
                def _():
                    pltpu.make_async_copy(ones_v, dacc.at[dv[k]],
                                          dsems[kb]).wait()

        @pl.loop(0, nchunks // NI)
        def _(t):
            for k in range(NI):
                i = t * NI + k

                @pl.when(i - 1 >= 0)
                def _():
                    wait_s((k - 1) % NI, (k - 1) % NBUF, (k - 1) % NBUF)

                    @pl.when(i - 1 + NI < nchunks)
                    def _():
                        issue_idx(i - 1 + NI, (k - 1) % NI)

                @pl.when(i + 1 < nchunks)
                def _():
                    wait_idx(i + 1, (k + 1) % NI)
                    issue_g(i + 1, (k + 1) % NI, (k + 1) % NBUF)

                wait_g(i, k % NBUF)
                issue_s(k, k % NBUF, k % NBUF)

        wait_s((nchunks - 1) % NI, (nchunks - 1) % NBUF, (nchunks - 1) % NBUF)

        plsc.subcore_barrier()

        ob = sid * zrows

        @pl.when(cid == 0)
        def _():
            pltpu.sync_copy(acc.at[pl.ds(ob, zrows)], out_lo.at[pl.ds(ob, zrows)])

        @pl.when(cid == 1)
        def _():
            pltpu.sync_copy(acc.at[pl.ds(ob, zrows)], out_hi.at[pl.ds(ob, zrows)])

        if with_deg:
            @pl.when(cid == 0)
            def _():
                pltpu.sync_copy(dacc.at[pl.ds(sid * zrows, zrows)],
                                deg_out.at[pl.ds(sid * zrows, zrows)])

    return pl.kernel(body, out_type=out_type, mesh=mesh, scratch_types=scratch)


def _selfproj_body(x_ref, w_ref, b_ref, out_ref):
    out_ref[...] = (jnp.dot(x_ref[...], w_ref[...],
                            preferred_element_type=jnp.float32)
                    + b_ref[...]).astype(out_ref.dtype)


def _mid_body(a_ref, slo_ref, shi_ref, deg_ref, wn1_ref, wn2_ref,
              h_ref, p_ref):
    invd = (1.0 / jnp.maximum(deg_ref[...], 1.0))[:, None]
    mean = jnp.concatenate([slo_ref[...], shi_ref[...]], axis=1) * invd
    h = jnp.maximum(
        a_ref[...].astype(jnp.float32)
        + jnp.dot(mean.astype(jnp.bfloat16), wn1_ref[...],
                  preferred_element_type=jnp.float32), 0.0)
    hb = h.astype(jnp.bfloat16)
    h_ref[...] = hb
    p_ref[...] = jnp.dot(hb, wn2_ref[...],
                         preferred_element_type=jnp.float32)


def _fin_body(s2_ref, slo_ref, shi_ref, deg_ref, out_ref):
    invd = (1.0 / jnp.maximum(deg_ref[...], 1.0))[:, None]
    out_ref[...] = s2_ref[...].astype(jnp.float32) + jnp.concatenate(
        [slo_ref[...], shi_ref[...]], axis=1) * invd


def kernel(x, edge_index, W_self1, W_neigh1, b1, W_self2, W_neigh2, b2):
    n, d_in = x.shape
    d_hid = W_self1.shape[1]
    d_out = W_self2.shape[1]
    e = edge_index.shape[1]
    assert d_in == 2 * HALF and d_out == 2 * HALF

    npad = (n // (NS * 16) + 1) * (NS * 16)
    epad = -(-e // (NS * CH * NBUF)) * (NS * CH * NBUF)
    pad = epad - e
    nchunks = epad // NS // CH
    src = edge_index[0]
    dst = edge_index[1]
    if pad:
        fill = jnp.arange(pad, dtype=jnp.int32)
        src = jnp.concatenate([src, fill % n])
        dst = jnp.concatenate([dst, n + fill % (npad - n)])
    segsum_deg = _make_segsum(n, npad, epad, True)
    segsum = _make_segsum(n, npad, epad, False)

    sum1lo, sum1hi, degp = segsum_deg(x, src, dst)

    bm = 1024
    grid = (-(-n // bm),)

    a = pl.pallas_call(
        _selfproj_body,
        grid=grid,
        in_specs=[
            pl.BlockSpec((bm, d_in), lambda i: (i, 0)),
            pl.BlockSpec((d_in, d_hid), lambda i: (0, 0)),
            pl.BlockSpec((1, d_hid), lambda i: (0, 0)),
        ],
        out_specs=pl.BlockSpec((bm, d_hid), lambda i: (i, 0)),
        out_shape=jax.ShapeDtypeStruct((n, d_hid), jnp.bfloat16),
    )(x, W_self1, b1.reshape(1, -1))

    h, p = pl.pallas_call(
        _mid_body,
        grid=grid,
        in_specs=[
            pl.BlockSpec((bm, d_hid), lambda i: (i, 0)),
            pl.BlockSpec((bm, HALF), lambda i: (i, 0)),
            pl.BlockSpec((bm, HALF), lambda i: (i, 0)),
            pl.BlockSpec((bm,), lambda i: (i,)),
            pl.BlockSpec((d_in, d_hid), lambda i: (0, 0)),
            pl.BlockSpec((d_hid, d_out), lambda i: (0, 0)),
        ],
        out_specs=[
            pl.BlockSpec((bm, d_hid), lambda i: (i, 0)),
            pl.BlockSpec((bm, d_out), lambda i: (i, 0)),
        ],
        out_shape=[
            jax.ShapeDtypeStruct((n, d_hid), jnp.bfloat16),
            jax.ShapeDtypeStruct((n, d_out), jnp.float32),
        ],
    )(a, sum1lo, sum1hi, degp, W_neigh1.astype(jnp.bfloat16),
      W_neigh2.astype(jnp.bfloat16))

    sum2lo, sum2hi = segsum(p, src, dst)

    s2 = pl.pallas_call(
        _selfproj_body,
        grid=grid,
        in_specs=[
            pl.BlockSpec((bm, d_hid), lambda i: (i, 0)),
            pl.BlockSpec((d_hid, d_out), lambda i: (0, 0)),
            pl.BlockSpec((1, d_out), lambda i: (0, 0)),
        ],
        out_specs=pl.BlockSpec((bm, d_out), lambda i: (i, 0)),
        out_shape=jax.ShapeDtypeStruct((n, d_out), jnp.bfloat16),
    )(h, W_self2.astype(jnp.bfloat16), b2.reshape(1, -1))

    out = pl.pallas_call(
        _fin_body,
        grid=grid,
        in_specs=[
            pl.BlockSpec((bm, d_out), lambda i: (i, 0)),
            pl.BlockSpec((bm, HALF), lambda i: (i, 0)),
            pl.BlockSpec((bm, HALF), lambda i: (i, 0)),
            pl.BlockSpec((bm,), lambda i: (i,)),
        ],
        out_specs=pl.BlockSpec((bm, d_out), lambda i: (i, 0)),
        out_shape=jax.ShapeDtypeStruct((n, d_out), jnp.float32),
    )(s2, sum2lo, sum2hi, degp)

    return out


# speedup vs baseline: 1.0371x; 1.0060x over previous
import functools

import jax
import jax.numpy as jnp
from jax import lax
from jax.experimental import pallas as pl
from jax.experimental.pallas import tpu as pltpu
from jax.experimental.pallas import tpu_sc as plsc

NC = 2
NS = 16
CH = 128
HALF = 128
NBUF = 2


def _make_segsum(n, npad, epad, with_deg):
    e_tile = epad // NS
    nchunks = e_tile // CH
    assert nchunks % NBUF == 0
    NI = 4
    zrows = npad // NS

    mesh = plsc.VectorSubcoreMesh(core_axis_name="c", subcore_axis_name="s")

    out_type = [
        jax.ShapeDtypeStruct((npad, HALF), jnp.float32),
        jax.ShapeDtypeStruct((npad, HALF), jnp.float32),
    ]
    scratch = [
        pltpu.VMEM((16, HALF), jnp.float32),
        pltpu.VMEM_SHARED((npad, HALF), jnp.float32),
    ]
    scratch += [pltpu.VMEM((CH,), jnp.int32) for _ in range(2 * NI)]
    scratch += [pltpu.SemaphoreType.DMA for _ in range(NI)]
    scratch += [pltpu.VMEM((CH, HALF), jnp.float32) for _ in range(NBUF)]
    scratch += [pltpu.SemaphoreType.DMA for _ in range(NBUF)]
    scratch += [pltpu.SemaphoreType.DMA for _ in range(NBUF)]
    scratch += [pltpu.SemaphoreType.DMA for _ in range(NBUF)]
    if with_deg:
        out_type.append(jax.ShapeDtypeStruct((npad,), jnp.float32))
        out_type.append(jax.ShapeDtypeStruct((npad,), jnp.float32))
        scratch += [
            pltpu.VMEM((CH,), jnp.float32),
            pltpu.VMEM((zrows,), jnp.float32),
            pltpu.VMEM_SHARED((npad,), jnp.float32),
        ]

    def body(tab, src_h, dst_h, out_lo, out_hi, *rest):
        if with_deg:
            deg0_out, deg1_out = rest[:2]
            rest = rest[2:]
        zb_v, acc = rest[:2]
        sv = rest[2:2 + NI]
        dv = rest[2 + NI:2 + 2 * NI]
        isems = rest[2 + 2 * NI:2 + 3 * NI]
        rows = rest[2 + 3 * NI:2 + 3 * NI + NBUF]
        gsems = rest[2 + 3 * NI + NBUF:2 + 3 * NI + 2 * NBUF]
        ssems = rest[2 + 3 * NI + 2 * NBUF:2 + 3 * NI + 3 * NBUF]
        dsems = rest[2 + 3 * NI + 3 * NBUF:2 + 3 * NI + 4 * NBUF]
        if with_deg:
            ones_v, zd_v, dacc = rest[2 + 3 * NI + 4 * NBUF:]
        cid = lax.axis_index("c")
        sid = lax.axis_index("s")

        zv = jnp.zeros((16,), jnp.float32)
        for r in range(16):
            for g in range(HALF // 16):
                zb_v[r, pl.ds(g * 16, 16)] = zv
        if with_deg:
            ov = jnp.full((16,), 1.0, jnp.float32)
            for g in range(CH // 16):
                ones_v[pl.ds(g * 16, 16)] = ov
            for g in range(zrows // 16):
                zd_v[pl.ds(g * 16, 16)] = zv

        sbase = sid * e_tile
        dbase = sid * e_tile
        coff = cid * HALF

        def issue_idx(i, k):
            pltpu.async_copy(src_h.at[pl.ds(sbase + i * CH, CH)], sv[k], isems[k])
            pltpu.async_copy(dst_h.at[pl.ds(dbase + i * CH, CH)], dv[k], isems[k])

        def wait_idx(i, k):
            pltpu.make_async_copy(src_h.at[pl.ds(sbase + i * CH, CH)], sv[k],
                                  isems[k]).wait()
            pltpu.make_async_copy(dst_h.at[pl.ds(dbase + i * CH, CH)], dv[k],
                                  isems[k]).wait()

        def issue_g(i, k, kr):
            pltpu.async_copy(tab.at[sv[k], pl.ds(coff, HALF)], rows[kr],
                             gsems[kr])

        def wait_g(i, kr):
            pltpu.make_async_copy(tab.at[sv[0], pl.ds(coff, HALF)], rows[kr],
                                  gsems[kr]).wait()

        for b in range(NI):
            issue_idx(b, b)

        @pl.loop(0, zrows // 16)
        def _(i):
            pltpu.sync_copy(zb_v, acc.at[pl.ds(sid * zrows + i * 16, 16)])

        if with_deg:
            pltpu.sync_copy(zd_v, dacc.at[pl.ds(sid * zrows, zrows)])

        wait_idx(0, 0)
        issue_g(0, 0, 0)
        plsc.subcore_barrier()

        hchunks = nchunks // 2

        def deg_pred(i):
            return ((cid == 0) & (i < hchunks)) | ((cid == 1) & (i >= hchunks))

        def issue_s(i, k, kr, kb):
            pltpu.async_copy(rows[kr], acc.at[dv[k]], ssems[kb], add=True)
            if with_deg:
                @pl.when(deg_pred(i))
                def _():
                    pltpu.async_copy(ones_v, dacc.at[dv[k]], dsems[kb], add=True)

        def wait_s(i, k, kr, kb):
            pltpu.make_async_copy(rows[kr], acc.at[dv[k]], ssems[kb]).wait()
            if with_deg:
                @pl.when(deg_pred(i))
                def _():
                    pltpu.make_async_copy(ones_v, dacc.at[dv[k]],
                                          dsems[kb]).wait()

        @pl.loop(0, nchunks // NI)
        def _(t):
            for k in range(NI):
                i = t * NI + k

                @pl.when(i - 1 >= 0)
                def _():
                    wait_s(i - 1, (k - 1) % NI, (k - 1) % NBUF, (k - 1) % NBUF)

                    @pl.when(i - 1 + NI < nchunks)
                    def _():
                        issue_idx(i - 1 + NI, (k - 1) % NI)

                @pl.when(i + 1 < nchunks)
                def _():
                    wait_idx(i + 1, (k + 1) % NI)
                    issue_g(i + 1, (k + 1) % NI, (k + 1) % NBUF)

                wait_g(i, k % NBUF)
                issue_s(i, k, k % NBUF, k % NBUF)

        wait_s(nchunks - 1, (nchunks - 1) % NI, (nchunks - 1) % NBUF,
               (nchunks - 1) % NBUF)

        plsc.subcore_barrier()

        ob = sid * zrows

        @pl.when(cid == 0)
        def _():
            pltpu.sync_copy(acc.at[pl.ds(ob, zrows)], out_lo.at[pl.ds(ob, zrows)])

        @pl.when(cid == 1)
        def _():
            pltpu.sync_copy(acc.at[pl.ds(ob, zrows)], out_hi.at[pl.ds(ob, zrows)])

        if with_deg:
            @pl.when(cid == 0)
            def _():
                pltpu.sync_copy(dacc.at[pl.ds(sid * zrows, zrows)],
                                deg0_out.at[pl.ds(sid * zrows, zrows)])

            @pl.when(cid == 1)
            def _():
                pltpu.sync_copy(dacc.at[pl.ds(sid * zrows, zrows)],
                                deg1_out.at[pl.ds(sid * zrows, zrows)])

    return pl.kernel(body, out_type=out_type, mesh=mesh, scratch_types=scratch)


def _selfproj_body(x_ref, w_ref, b_ref, out_ref):
    out_ref[...] = (jnp.dot(x_ref[...], w_ref[...],
                            preferred_element_type=jnp.float32)
                    + b_ref[...]).astype(out_ref.dtype)


def _mid_body(a_ref, slo_ref, shi_ref, d0_ref, d1_ref, wn1_ref, wn2_ref,
              h_ref, p_ref):
    invd = (1.0 / jnp.maximum(d0_ref[...] + d1_ref[...], 1.0))[:, None]
    mean = jnp.concatenate([slo_ref[...], shi_ref[...]], axis=1) * invd
    h = jnp.maximum(
        a_ref[...].astype(jnp.float32)
        + jnp.dot(mean.astype(jnp.bfloat16), wn1_ref[...],
                  preferred_element_type=jnp.float32), 0.0)
    hb = h.astype(jnp.bfloat16)
    h_ref[...] = hb
    p_ref[...] = jnp.dot(hb, wn2_ref[...],
                         preferred_element_type=jnp.float32)


def _fin_body(s2_ref, slo_ref, shi_ref, d0_ref, d1_ref, out_ref):
    invd = (1.0 / jnp.maximum(d0_ref[...] + d1_ref[...], 1.0))[:, None]
    out_ref[...] = s2_ref[...].astype(jnp.float32) + jnp.concatenate(
        [slo_ref[...], shi_ref[...]], axis=1) * invd


def kernel(x, edge_index, W_self1, W_neigh1, b1, W_self2, W_neigh2, b2):
    n, d_in = x.shape
    d_hid = W_self1.shape[1]
    d_out = W_self2.shape[1]
    e = edge_index.shape[1]
    assert d_in == 2 * HALF and d_out == 2 * HALF

    npad = (n // (NS * 16) + 1) * (NS * 16)
    epad = -(-e // (NS * CH * NBUF)) * (NS * CH * NBUF)
    pad = epad - e
    nchunks = epad // NS // CH
    src = edge_index[0]
    dst = edge_index[1]
    if pad:
        fill = jnp.arange(pad, dtype=jnp.int32)
        src = jnp.concatenate([src, fill % n])
        dst = jnp.concatenate([dst, n + fill % (npad - n)])
    segsum_deg = _make_segsum(n, npad, epad, True)
    segsum = _make_segsum(n, npad, epad, False)

    sum1lo, sum1hi, deg0, deg1 = segsum_deg(x, src, dst)

    bm = 1024
    grid = (-(-n // bm),)

    a = pl.pallas_call(
        _selfproj_body,
        grid=grid,
        in_specs=[
            pl.BlockSpec((bm, d_in), lambda i: (i, 0)),
            pl.BlockSpec((d_in, d_hid), lambda i: (0, 0)),
            pl.BlockSpec((1, d_hid), lambda i: (0, 0)),
        ],
        out_specs=pl.BlockSpec((bm, d_hid), lambda i: (i, 0)),
        out_shape=jax.ShapeDtypeStruct((n, d_hid), jnp.bfloat16),
    )(x, W_self1, b1.reshape(1, -1))

    h, p = pl.pallas_call(
        _mid_body,
        grid=grid,
        in_specs=[
            pl.BlockSpec((bm, d_hid), lambda i: (i, 0)),
            pl.BlockSpec((bm, HALF), lambda i: (i, 0)),
            pl.BlockSpec((bm, HALF), lambda i: (i, 0)),
            pl.BlockSpec((bm,), lambda i: (i,)),
            pl.BlockSpec((bm,), lambda i: (i,)),
            pl.BlockSpec((d_in, d_hid), lambda i: (0, 0)),
            pl.BlockSpec((d_hid, d_out), lambda i: (0, 0)),
        ],
        out_specs=[
            pl.BlockSpec((bm, d_hid), lambda i: (i, 0)),
            pl.BlockSpec((bm, d_out), lambda i: (i, 0)),
        ],
        out_shape=[
            jax.ShapeDtypeStruct((n, d_hid), jnp.bfloat16),
            jax.ShapeDtypeStruct((n, d_out), jnp.float32),
        ],
    )(a, sum1lo, sum1hi, deg0, deg1, W_neigh1.astype(jnp.bfloat16),
      W_neigh2.astype(jnp.bfloat16))

    sum2lo, sum2hi = segsum(p, src, dst)

    s2 = pl.pallas_call(
        _selfproj_body,
        grid=grid,
        in_specs=[
            pl.BlockSpec((bm, d_hid), lambda i: (i, 0)),
            pl.BlockSpec((d_hid, d_out), lambda i: (0, 0)),
            pl.BlockSpec((1, d_out), lambda i: (0, 0)),
        ],
        out_specs=pl.BlockSpec((bm, d_out), lambda i: (i, 0)),
        out_shape=jax.ShapeDtypeStruct((n, d_out), jnp.bfloat16),
    )(h, W_self2.astype(jnp.bfloat16), b2.reshape(1, -1))

    out = pl.pallas_call(
        _fin_body,
        grid=grid,
        in_specs=[
            pl.BlockSpec((bm, d_out), lambda i: (i, 0)),
            pl.BlockSpec((bm, HALF), lambda i: (i, 0)),
            pl.BlockSpec((bm, HALF), lambda i: (i, 0)),
            pl.BlockSpec((bm,), lambda i: (i,)),
            pl.BlockSpec((bm,), lambda i: (i,)),
        ],
        out_specs=pl.BlockSpec((bm, d_out), lambda i: (i, 0)),
        out_shape=jax.ShapeDtypeStruct((n, d_out), jnp.float32),
    )(s2, sum2lo, sum2hi, deg0, deg1)

    return out


# speedup vs baseline: 1.0395x; 1.0023x over previous
import functools

import jax
import jax.numpy as jnp
from jax import lax
from jax.experimental import pallas as pl
from jax.experimental.pallas import tpu as pltpu
from jax.experimental.pallas import tpu_sc as plsc

NC = 2
NS = 16
CH = 128
HALF = 128
NBUF = 2


def _make_segsum(n, npad, e, with_deg):
    assert e % NS == 0
    e_tile = e // NS
    nchunks = e_tile // CH
    tail = e_tile - nchunks * CH
    NI = 6
    assert nchunks % NI == 0 and NI % NBUF == 0
    zrows = npad // NS

    mesh = plsc.VectorSubcoreMesh(core_axis_name="c", subcore_axis_name="s")

    out_type = [
        jax.ShapeDtypeStruct((npad, HALF), jnp.float32),
        jax.ShapeDtypeStruct((npad, HALF), jnp.float32),
    ]
    scratch = [
        pltpu.VMEM((16, HALF), jnp.float32),
        pltpu.VMEM_SHARED((npad, HALF), jnp.float32),
    ]
    scratch += [pltpu.VMEM((CH,), jnp.int32) for _ in range(2 * NI)]
    scratch += [pltpu.SemaphoreType.DMA for _ in range(NI)]
    scratch += [pltpu.VMEM((CH, HALF), jnp.float32) for _ in range(NBUF)]
    scratch += [pltpu.SemaphoreType.DMA for _ in range(NBUF)]
    scratch += [pltpu.SemaphoreType.DMA for _ in range(NBUF)]
    scratch += [pltpu.SemaphoreType.DMA for _ in range(NBUF)]
    if with_deg:
        out_type.append(jax.ShapeDtypeStruct((npad,), jnp.float32))
        out_type.append(jax.ShapeDtypeStruct((npad,), jnp.float32))
        scratch += [
            pltpu.VMEM((CH,), jnp.float32),
            pltpu.VMEM((zrows,), jnp.float32),
            pltpu.VMEM_SHARED((npad,), jnp.float32),
        ]
    if tail:
        scratch += [
            pltpu.VMEM((tail,), jnp.int32),
            pltpu.VMEM((tail,), jnp.int32),
            pltpu.VMEM((tail, HALF), jnp.float32),
            pltpu.SemaphoreType.DMA,
        ]

    def body(tab, src_h, dst_h, out_lo, out_hi, *rest):
        if with_deg:
            deg0_out, deg1_out = rest[:2]
            rest = rest[2:]
        zb_v, acc = rest[:2]
        sv = rest[2:2 + NI]
        dv = rest[2 + NI:2 + 2 * NI]
        isems = rest[2 + 2 * NI:2 + 3 * NI]
        rows = rest[2 + 3 * NI:2 + 3 * NI + NBUF]
        gsems = rest[2 + 3 * NI + NBUF:2 + 3 * NI + 2 * NBUF]
        ssems = rest[2 + 3 * NI + 2 * NBUF:2 + 3 * NI + 3 * NBUF]
        dsems = rest[2 + 3 * NI + 3 * NBUF:2 + 3 * NI + 4 * NBUF]
        pos = 2 + 3 * NI + 4 * NBUF
        if with_deg:
            ones_v, zd_v, dacc = rest[pos:pos + 3]
            pos += 3
        if tail:
            sv_t, dv_t, rows_t, isem_t = rest[pos:pos + 4]
        cid = lax.axis_index("c")
        sid = lax.axis_index("s")

        zv = jnp.zeros((16,), jnp.float32)
        for r in range(16):
            for g in range(HALF // 16):
                zb_v[r, pl.ds(g * 16, 16)] = zv
        if with_deg:
            ov = jnp.full((16,), 1.0, jnp.float32)
            for g in range(CH // 16):
                ones_v[pl.ds(g * 16, 16)] = ov
            for g in range(zrows // 16):
                zd_v[pl.ds(g * 16, 16)] = zv

        sbase = sid * e_tile
        dbase = sid * e_tile
        coff = cid * HALF

        def issue_idx(i, k):
            pltpu.async_copy(src_h.at[pl.ds(sbase + i * CH, CH)], sv[k], isems[k])
            pltpu.async_copy(dst_h.at[pl.ds(dbase + i * CH, CH)], dv[k], isems[k])

        def wait_idx(i, k):
            pltpu.make_async_copy(src_h.at[pl.ds(sbase + i * CH, CH)], sv[k],
                                  isems[k]).wait()
            pltpu.make_async_copy(dst_h.at[pl.ds(dbase + i * CH, CH)], dv[k],
                                  isems[k]).wait()

        def issue_g(i, k, kr):
            pltpu.async_copy(tab.at[sv[k], pl.ds(coff, HALF)], rows[kr],
                             gsems[kr])

        def wait_g(i, kr):
            pltpu.make_async_copy(tab.at[sv[0], pl.ds(coff, HALF)], rows[kr],
                                  gsems[kr]).wait()

        for b in range(NI):
            issue_idx(b, b)
        if tail:
            pltpu.async_copy(src_h.at[pl.ds(sbase + nchunks * CH, tail)],
                             sv_t, isem_t)
            pltpu.async_copy(dst_h.at[pl.ds(dbase + nchunks * CH, tail)],
                             dv_t, isem_t)

        @pl.loop(0, zrows // 16)
        def _(i):
            pltpu.sync_copy(zb_v, acc.at[pl.ds(sid * zrows + i * 16, 16)])

        if with_deg:
            pltpu.sync_copy(zd_v, dacc.at[pl.ds(sid * zrows, zrows)])

        wait_idx(0, 0)
        issue_g(0, 0, 0)
        plsc.subcore_barrier()

        hchunks = nchunks // 2

        def deg_pred(i):
            return ((cid == 0) & (i < hchunks)) | ((cid == 1) & (i >= hchunks))

        def issue_s(i, k, kr, kb):
            pltpu.async_copy(rows[kr], acc.at[dv[k]], ssems[kb], add=True)
            if with_deg:
                @pl.when(deg_pred(i))
                def _():
                    pltpu.async_copy(ones_v, dacc.at[dv[k]], dsems[kb], add=True)

        def wait_s(i, k, kr, kb):
            pltpu.make_async_copy(rows[kr], acc.at[dv[k]], ssems[kb]).wait()
            if with_deg:
                @pl.when(deg_pred(i))
                def _():
                    pltpu.make_async_copy(ones_v, dacc.at[dv[k]],
                                          dsems[kb]).wait()

        @pl.loop(0, nchunks // NI)
        def _(t):
            for k in range(NI):
                i = t * NI + k

                @pl.when(i - 1 >= 0)
                def _():
                    wait_s(i - 1, (k - 1) % NI, (k - 1) % NBUF, (k - 1) % NBUF)

                    @pl.when(i - 1 + NI < nchunks)
                    def _():
                        issue_idx(i - 1 + NI, (k - 1) % NI)

                @pl.when(i + 1 < nchunks)
                def _():
                    wait_idx(i + 1, (k + 1) % NI)
                    issue_g(i + 1, (k + 1) % NI, (k + 1) % NBUF)

                wait_g(i, k % NBUF)
                issue_s(i, k, k % NBUF, k % NBUF)

        wait_s(nchunks - 1, (nchunks - 1) % NI, (nchunks - 1) % NBUF,
               (nchunks - 1) % NBUF)

        if tail:
            pltpu.make_async_copy(
                src_h.at[pl.ds(sbase + nchunks * CH, tail)], sv_t,
                isem_t).wait()
            pltpu.make_async_copy(
                dst_h.at[pl.ds(dbase + nchunks * CH, tail)], dv_t,
                isem_t).wait()
            pltpu.async_copy(tab.at[sv_t, pl.ds(coff, HALF)], rows_t,
                             gsems[0])
            pltpu.make_async_copy(tab.at[sv_t, pl.ds(coff, HALF)], rows_t,
                                  gsems[0]).wait()
            pltpu.async_copy(rows_t, acc.at[dv_t], ssems[0], add=True)
            pltpu.make_async_copy(rows_t, acc.at[dv_t], ssems[0]).wait()
            if with_deg:
                @pl.when(cid == 0)
                def _():
                    pltpu.async_copy(ones_v.at[pl.ds(0, tail)],
                                     dacc.at[dv_t], dsems[0], add=True)
                    pltpu.make_async_copy(ones_v.at[pl.ds(0, tail)],
                                          dacc.at[dv_t], dsems[0]).wait()

        plsc.subcore_barrier()

        ob = sid * zrows

        @pl.when(cid == 0)
        def _():
            pltpu.sync_copy(acc.at[pl.ds(ob, zrows)], out_lo.at[pl.ds(ob, zrows)])

        @pl.when(cid == 1)
        def _():
            pltpu.sync_copy(acc.at[pl.ds(ob, zrows)], out_hi.at[pl.ds(ob, zrows)])

        if with_deg:
            @pl.when(cid == 0)
            def _():
                pltpu.sync_copy(dacc.at[pl.ds(sid * zrows, zrows)],
                                deg0_out.at[pl.ds(sid * zrows, zrows)])

            @pl.when(cid == 1)
            def _():
                pltpu.sync_copy(dacc.at[pl.ds(sid * zrows, zrows)],
                                deg1_out.at[pl.ds(sid * zrows, zrows)])

    return pl.kernel(body, out_type=out_type, mesh=mesh, scratch_types=scratch)


def _selfproj_body(x_ref, w_ref, b_ref, out_ref):
    out_ref[...] = (jnp.dot(x_ref[...], w_ref[...],
                            preferred_element_type=jnp.float32)
                    + b_ref[...]).astype(out_ref.dtype)


def _mid_body(a_ref, slo_ref, shi_ref, d0_ref, d1_ref, wn1_ref, wn2_ref,
              h_ref, p_ref):
    invd = (1.0 / jnp.maximum(d0_ref[...] + d1_ref[...], 1.0))[:, None]
    mean = jnp.concatenate([slo_ref[...], shi_ref[...]], axis=1) * invd
    h = jnp.maximum(
        a_ref[...].astype(jnp.float32)
        + jnp.dot(mean.astype(jnp.bfloat16), wn1_ref[...],
                  preferred_element_type=jnp.float32), 0.0)
    hb = h.astype(jnp.bfloat16)
    h_ref[...] = hb
    p_ref[...] = jnp.dot(hb, wn2_ref[...],
                         preferred_element_type=jnp.float32)


def _fin_body(s2_ref, slo_ref, shi_ref, d0_ref, d1_ref, out_ref):
    invd = (1.0 / jnp.maximum(d0_ref[...] + d1_ref[...], 1.0))[:, None]
    out_ref[...] = s2_ref[...].astype(jnp.float32) + jnp.concatenate(
        [slo_ref[...], shi_ref[...]], axis=1) * invd


def kernel(x, edge_index, W_self1, W_neigh1, b1, W_self2, W_neigh2, b2):
    n, d_in = x.shape
    d_hid = W_self1.shape[1]
    d_out = W_self2.shape[1]
    e = edge_index.shape[1]
    assert d_in == 2 * HALF and d_out == 2 * HALF

    npad = (n // (NS * 16) + 1) * (NS * 16)
    src = edge_index[0]
    dst = edge_index[1]
    segsum_deg = _make_segsum(n, npad, e, True)
    segsum = _make_segsum(n, npad, e, False)

    sum1lo, sum1hi, deg0, deg1 = segsum_deg(x, src, dst)

    bm = 1024
    grid = (-(-n // bm),)

    a = pl.pallas_call(
        _selfproj_body,
        grid=grid,
        in_specs=[
            pl.BlockSpec((bm, d_in), lambda i: (i, 0)),
            pl.BlockSpec((d_in, d_hid), lambda i: (0, 0)),
            pl.BlockSpec((1, d_hid), lambda i: (0, 0)),
        ],
        out_specs=pl.BlockSpec((bm, d_hid), lambda i: (i, 0)),
        out_shape=jax.ShapeDtypeStruct((n, d_hid), jnp.bfloat16),
    )(x, W_self1, b1.reshape(1, -1))

    h, p = pl.pallas_call(
        _mid_body,
        grid=grid,
        in_specs=[
            pl.BlockSpec((bm, d_hid), lambda i: (i, 0)),
            pl.BlockSpec((bm, HALF), lambda i: (i, 0)),
            pl.BlockSpec((bm, HALF), lambda i: (i, 0)),
            pl.BlockSpec((bm,), lambda i: (i,)),
            pl.BlockSpec((bm,), lambda i: (i,)),
            pl.BlockSpec((d_in, d_hid), lambda i: (0, 0)),
            pl.BlockSpec((d_hid, d_out), lambda i: (0, 0)),
        ],
        out_specs=[
            pl.BlockSpec((bm, d_hid), lambda i: (i, 0)),
            pl.BlockSpec((bm, d_out), lambda i: (i, 0)),
        ],
        out_shape=[
            jax.ShapeDtypeStruct((n, d_hid), jnp.bfloat16),
            jax.ShapeDtypeStruct((n, d_out), jnp.float32),
        ],
    )(a, sum1lo, sum1hi, deg0, deg1, W_neigh1.astype(jnp.bfloat16),
      W_neigh2.astype(jnp.bfloat16))

    sum2lo, sum2hi = segsum(p, src, dst)

    s2 = pl.pallas_call(
        _selfproj_body,
        grid=grid,
        in_specs=[
            pl.BlockSpec((bm, d_hid), lambda i: (i, 0)),
            pl.BlockSpec((d_hid, d_out), lambda i: (0, 0)),
            pl.BlockSpec((1, d_out), lambda i: (0, 0)),
        ],
        out_specs=pl.BlockSpec((bm, d_out), lambda i: (i, 0)),
        out_shape=jax.ShapeDtypeStruct((n, d_out), jnp.bfloat16),
    )(h, W_self2.astype(jnp.bfloat16), b2.reshape(1, -1))

    out = pl.pallas_call(
        _fin_body,
        grid=grid,
        in_specs=[
            pl.BlockSpec((bm, d_out), lambda i: (i, 0)),
            pl.BlockSpec((bm, HALF), lambda i: (i, 0)),
            pl.BlockSpec((bm, HALF), lambda i: (i, 0)),
            pl.BlockSpec((bm,), lambda i: (i,)),
            pl.BlockSpec((bm,), lambda i: (i,)),
        ],
        out_specs=pl.BlockSpec((bm, d_out), lambda i: (i, 0)),
        out_shape=jax.ShapeDtypeStruct((n, d_out), jnp.float32),
    )(s2, sum2lo, sum2hi, deg0, deg1)

    return out
